# baseline (speedup 1.0000x reference)
import functools

import jax
import jax.numpy as jnp
from jax import lax
from jax.experimental import pallas as pl
from jax.experimental.pallas import tpu as pltpu
from jax.experimental.pallas import tpu_sc as plsc

N = 50000
E = 1600000
NC = 2
NS = 16
NT = NC * NS
EPT = E // NT
B1 = 2048
NCH1 = 24
B2 = 4096
NCH2 = 12
TAIL = 848
PTW = 25008
NPAD = 50176
SLICE = NPAD // NS
SV = SLICE // 16
DIAG_T = 25
DIAG_B = N // DIAG_T

_f32 = jnp.float32
_i32 = jnp.int32


@functools.cache
def _mesh():
    return plsc.VectorSubcoreMesh(
        core_axis_name="c", subcore_axis_name="s",
        num_cores=NC, num_subcores=NS)


def _rsqrt_newton(x):
    i = plsc.bitcast(x, _i32)
    y = plsc.bitcast(jnp.int32(0x5F3759DF) - (i >> 1), _f32)
    for _ in range(3):
        y = y * (1.5 - 0.5 * x * y * y)
    return jnp.where(x > 0.0, y, 0.0)


def _issue_first(in_descs, bb):
    for d in in_descs(0, 0, bb):
        d.start()
    for d in in_descs(1, 1, bb):
        d.start()


def _chunk_pipeline(in_descs, out_desc, compute, bb, nchb):
    def issue_in(slot, k, n):
        for d in in_descs(slot, k, n):
            d.start()

    def wait_in(slot, k, n):
        for d in in_descs(slot, k, n):
            d.wait()

    def outer(g, c):
        for b in range(2):
            k = 2 * g + b
            wait_in(b, k, bb)

            @pl.when(k >= 2)
            def _():
                out_desc(b, k - 2, bb).wait()
            compute(b, bb // 16)
            out_desc(b, k, bb).start()

            @pl.when(k < nchb - 2)
            def _():
                issue_in(b, k + 2, bb)

            @pl.when(k == nchb - 2)
            def _():
                issue_in(0, nchb, TAIL)
        return c
    lax.fori_loop(0, nchb // 2, outer, 0)

    wait_in(0, nchb, TAIL)
    out_desc(0, nchb - 2, bb).wait()
    compute(0, TAIL // 16)
    out_desc(0, nchb, TAIL).start()
    out_desc(1, nchb - 1, bb).wait()
    out_desc(0, nchb, TAIL).wait()


def _k1_graph(conf_h, imp_h, pca_h, head_h, vals_h, part_h,
              acc, stage, rs, rowbufs, cbufs, ibufs, pbufs, hbufs, vbufs,
              isems, osems, rsems, wid, sid, cid):
    base = wid * EPT

    def in_descs(slot, k, n):
        off = base + k * B1
        dst = pl.ds(0, n)
        return (
            pltpu.make_async_copy(conf_h.at[pl.ds(off, n)], cbufs[slot].at[dst], isems[slot]),
            pltpu.make_async_copy(imp_h.at[pl.ds(off, n)], ibufs[slot].at[dst], isems[slot]),
            pltpu.make_async_copy(pca_h.at[pl.ds(off, n)], pbufs[slot].at[dst], isems[slot]),
            pltpu.make_async_copy(head_h.at[pl.ds(off, n)], hbufs[slot].at[dst], isems[slot]),
        )

    def out_desc(slot, k, n):
        wn = 432 if n == TAIL else n // 2
        return pltpu.make_async_copy(
            vbufs[slot].at[pl.ds(0, wn)],
            vals_h.at[pl.ds(wid * PTW + k * (B1 // 2), wn)], osems[slot])

    def compute(slot, nvec):
        cb, ib, pb, hb, vb = (cbufs[slot], ibufs[slot], pbufs[slot],
                              hbufs[slot], vbufs[slot])

        def val_vec(sl):
            v = cb[sl] * ib[sl] * pb[sl]
            plsc.addupdate_scatter(acc, [hb[sl]], v)
            return v

        def one_pair(p):
            v0 = val_vec(pl.ds(p * 32, 16))
            v1 = val_vec(pl.ds(p * 32 + 16, 16))
            w = lax.shift_right_logical(plsc.bitcast(v0, _i32), 16) | (
                plsc.bitcast(v1, _i32) & jnp.int32(-65536))
            vb[pl.ds(p * 16, 16)] = w

        npair = nvec // 2
        if npair % 4 == 0:
            def vec_body(i, c):
                for u in range(4):
                    one_pair(i * 4 + u)
                return c
            lax.fori_loop(0, npair // 4, vec_body, 0)
        else:
            def vec_body(i, c):
                one_pair(i)
                return c
            lax.fori_loop(0, npair, vec_body, 0)
        if nvec % 2:
            v0 = val_vec(pl.ds(npair * 32, 16))
            vb[pl.ds(npair * 16, 16)] = lax.shift_right_logical(
                plsc.bitcast(v0, _i32), 16)

    def edges():
        with jax.named_scope("k1_edges"):
            _chunk_pipeline(in_descs, out_desc, compute, B1, NCH1)

    def stage_fn():
        with jax.named_scope("k1_stage"):
            pltpu.sync_copy(acc, stage.at[pl.ds(sid * NPAD, NPAD)])
            plsc.subcore_barrier()

    def reduce_fn():
        with jax.named_scope("k1_reduce"):
            soff = sid * SLICE

            def zrs_body(i, c):
                for u in range(7):
                    rs[pl.ds((i * 7 + u) * 16, 16)] = jnp.zeros((16,), _f32)
                return c
            lax.fori_loop(0, SV // 7, zrs_body, 0)

            def row_desc(slot, r):
                return pltpu.make_async_copy(
                    stage.at[pl.ds(r * NPAD + soff, SLICE)],
                    rowbufs[slot], rsems[slot])

            row_desc(0, 0).start()
            row_desc(1, 1).start()

            def row_outer(g, c):
                for b in range(2):
                    r = 2 * g + b
                    row_desc(b, r).wait()
                    rb = rowbufs[b]

                    def add_body(i, c2):
                        for u in range(7):
                            t = pl.ds((i * 7 + u) * 16, 16)
                            rs[t] = rs[t] + rb[t]
                        return c2
                    lax.fori_loop(0, SV // 7, add_body, 0)

                    @pl.when(r < NS - 2)
                    def _():
                        row_desc(b, r + 2).start()
                return c
            lax.fori_loop(0, NS // 2, row_outer, 0)
            pltpu.sync_copy(rs, part_h.at[pl.ds(cid * NPAD + soff, SLICE)])
            plsc.subcore_barrier()

    return in_descs, edges, stage_fn, reduce_fn


def _k1_zero_acc(acc):
    with jax.named_scope("k1_zero"):
        def zero_body(i, c):
            for u in range(8):
                acc[pl.ds((i * 8 + u) * 16, 16)] = jnp.zeros((16,), _f32)
            return c
        lax.fori_loop(0, NPAD // 16 // 8, zero_body, 0)


def _k1_body(conf_sr, imp_sr, pca_sr, head_sr,
             conf_tg, imp_tg, pca_tg, head_tg,
             vals_sr, vals_tg, part_sr, part_tg,
             acc, stage, rs, rb0, rb1, cb0, cb1, ib0, ib1, pb0, pb1,
             hb0, hb1, vb0, vb1,
             isem0, isem1, osem0, osem1, rsem0, rsem1):
    sid = lax.axis_index("s")
    cid = lax.axis_index("c")
    wid = sid * NC + cid
    args = (acc, stage, rs, (rb0, rb1), (cb0, cb1), (ib0, ib1), (pb0, pb1),
            (hb0, hb1), (vb0, vb1), (isem0, isem1), (osem0, osem1),
            (rsem0, rsem1), wid, sid, cid)
    in_sr, edges_sr, stage_sr, reduce_sr = _k1_graph(
        conf_sr, imp_sr, pca_sr, head_sr, vals_sr, part_sr, *args)
    in_tg, edges_tg, stage_tg, reduce_tg = _k1_graph(
        conf_tg, imp_tg, pca_tg, head_tg, vals_tg, part_tg, *args)
    _issue_first(in_sr, B1)
    _k1_zero_acc(acc)
    edges_sr()
    stage_sr()
    _issue_first(in_tg, B1)
    _k1_zero_acc(acc)
    reduce_sr()
    edges_tg()
    stage_tg()
    reduce_tg()


@functools.cache
def _k1():
    return functools.partial(
        pl.kernel,
        out_type=(
            jax.ShapeDtypeStruct((NT * PTW,), _i32),
            jax.ShapeDtypeStruct((NT * PTW,), _i32),
            jax.ShapeDtypeStruct((NC * NPAD,), _f32),
            jax.ShapeDtypeStruct((NC * NPAD,), _f32),
        ),
        mesh=_mesh(),
        compiler_params=pltpu.CompilerParams(needs_layout_passes=False),
        scratch_types=(
            pltpu.VMEM((NPAD,), _f32),
            pltpu.VMEM_SHARED((NS * NPAD,), _f32),
            pltpu.VMEM((SLICE,), _f32),
            pltpu.VMEM((SLICE,), _f32), pltpu.VMEM((SLICE,), _f32),
            pltpu.VMEM((B1,), _f32), pltpu.VMEM((B1,), _f32),
            pltpu.VMEM((B1,), _f32), pltpu.VMEM((B1,), _f32),
            pltpu.VMEM((B1,), _f32), pltpu.VMEM((B1,), _f32),
            pltpu.VMEM((B1,), _i32), pltpu.VMEM((B1,), _i32),
            pltpu.VMEM((B1 // 2,), _i32), pltpu.VMEM((B1 // 2,), _i32),
            pltpu.SemaphoreType.DMA, pltpu.SemaphoreType.DMA,
            pltpu.SemaphoreType.DMA, pltpu.SemaphoreType.DMA,
            pltpu.SemaphoreType.DMA, pltpu.SemaphoreType.DMA,
        ),
    )(_k1_body)


def _k2_graph(head_h, tail_h, vals_h, part_h, adj_h,
              d_sh, d_ref, rs, rowbufs, hbufs, tbufs, vbufs, obufs,
              rsems, isems, osems, sid, wid):
    base = wid * EPT

    def in_descs(slot, k, n):
        off = base + k * B2
        dst = pl.ds(0, n)
        return (
            pltpu.make_async_copy(head_h.at[pl.ds(off, n)], hbufs[slot].at[dst], isems[slot]),
            pltpu.make_async_copy(tail_h.at[pl.ds(off, n)], tbufs[slot].at[dst], isems[slot]),
            pltpu.make_async_copy(
                vals_h.at[pl.ds(wid * PTW + k * (B2 // 2),
                                432 if n == TAIL else n // 2)],
                vbufs[slot].at[pl.ds(0, 432 if n == TAIL else n // 2)],
                isems[slot]),
        )

    _issue_first(in_descs, B2)

    soff = sid * SLICE

    with jax.named_scope("k2_init"):
        def one_body(i, c):
            for u in range(7):
                rs[pl.ds((i * 7 + u) * 16, 16)] = jnp.full((16,), 1.0, _f32)
            return c
        lax.fori_loop(0, SV // 7, one_body, 0)

    def row_desc(slot, r):
        return pltpu.make_async_copy(
            part_h.at[pl.ds(r * NPAD + soff, SLICE)], rowbufs[slot], rsems[slot])

    row_desc(0, 0).start()
    row_desc(1, 1).start()
    with jax.named_scope("k2_reduce"):
        for b in range(NC):
            row_desc(b, b).wait()
            rb = rowbufs[b]

            def add_body(i, c2):
                for u in range(7):
                    t = pl.ds((i * 7 + u) * 16, 16)
                    rs[t] = rs[t] + rb[t]
                return c2
            lax.fori_loop(0, SV // 7, add_body, 0)

    with jax.named_scope("k2_newton"):
        def newton_body(i, c):
            for u in range(4):
                s = pl.ds((i * 4 + u) * 16, 16)
                rowbufs[0][s] = _rsqrt_newton(rs[s])
            return c
        lax.fori_loop(0, SV // 4, newton_body, 0)
        pltpu.sync_copy(rowbufs[0], d_sh.at[pl.ds(soff, SLICE)])
    with jax.named_scope("k2_barrier"):
        plsc.subcore_barrier()

    with jax.named_scope("k2_dcopy"):
        pltpu.sync_copy(d_sh, d_ref)

    def out_desc(slot, k, n):
        return pltpu.make_async_copy(
            obufs[slot].at[pl.ds(0, n)],
            adj_h.at[pl.ds(base + k * B2, n)], osems[slot])

    def compute(slot, nvec):
        hb, tb, vb, ob = hbufs[slot], tbufs[slot], vbufs[slot], obufs[slot]

        def scale(sl, v):
            dh = plsc.load_gather(d_ref, [hb[sl]])
            dt = plsc.load_gather(d_ref, [tb[sl]])
            ob[sl] = v * dh * dt

        def one_pair(p):
            w = vb[pl.ds(p * 16, 16)]
            va = plsc.bitcast(lax.shift_left(w, 16), _f32)
            vb1 = plsc.bitcast(w & jnp.int32(-65536), _f32)
            scale(pl.ds(p * 32, 16), va)
            scale(pl.ds(p * 32 + 16, 16), vb1)

        npair = nvec // 2
        if npair % 4 == 0:
            def vec_body(i, c):
                for u in range(4):
                    one_pair(i * 4 + u)
                return c
            lax.fori_loop(0, npair // 4, vec_body, 0)
        else:
            def vec_body(i, c):
                one_pair(i)
                return c
            lax.fori_loop(0, npair, vec_body, 0)
        if nvec % 2:
            w = vb[pl.ds(npair * 16, 16)]
            va = plsc.bitcast(lax.shift_left(w, 16), _f32)
            scale(pl.ds(npair * 32, 16), va)

    with jax.named_scope("k2_gather"):
        _chunk_pipeline(in_descs, out_desc, compute, B2, NCH2)

    with jax.named_scope("k2_diag"):
        @pl.when(wid < DIAG_T)
        def _():
            doff = wid * DIAG_B
            ob = obufs[0]

            def diag_body(i, c):
                for u in range(5):
                    j = i * 5 + u
                    y = d_ref[pl.ds(doff + j * 16, 16)]
                    ob[pl.ds(j * 16, 16)] = y * y
                return c
            lax.fori_loop(0, DIAG_B // 16 // 5, diag_body, 0)
            pltpu.sync_copy(ob.at[pl.ds(0, DIAG_B)],
                            adj_h.at[pl.ds(E + doff, DIAG_B)])


def _k2_body(head_sr, tail_sr, vals_sr, part_sr,
             head_tg, tail_tg, vals_tg, part_tg,
             adj_sr, adj_tg,
             d_sh, d_ref, rs, rb0, rb1, hb0, hb1, tb0, tb1, vb0, vb1,
             ob0, ob1, rsem0, rsem1, isem0, isem1, osem0, osem1):
    sid = lax.axis_index("s")
    wid = sid * NC + lax.axis_index("c")
    args = (d_sh, d_ref, rs, (rb0, rb1), (hb0, hb1), (tb0, tb1), (vb0, vb1),
            (ob0, ob1), (rsem0, rsem1), (isem0, isem1), (osem0, osem1),
            sid, wid)
    _k2_graph(head_sr, tail_sr, vals_sr, part_sr, adj_sr, *args)
    plsc.subcore_barrier()
    _k2_graph(head_tg, tail_tg, vals_tg, part_tg, adj_tg, *args)


@functools.cache
def _k2():
    return functools.partial(
        pl.kernel,
        out_type=(
            jax.ShapeDtypeStruct((E + N,), _f32),
            jax.ShapeDtypeStruct((E + N,), _f32),
        ),
        mesh=_mesh(),
        compiler_params=pltpu.CompilerParams(needs_layout_passes=False),
        scratch_types=(
            pltpu.VMEM_SHARED((NPAD,), _f32),
            pltpu.VMEM((NPAD,), _f32),
            pltpu.VMEM((SLICE,), _f32),
            pltpu.VMEM((SLICE,), _f32), pltpu.VMEM((SLICE,), _f32),
            pltpu.VMEM((B2,), _i32), pltpu.VMEM((B2,), _i32),
            pltpu.VMEM((B2,), _i32), pltpu.VMEM((B2,), _i32),
            pltpu.VMEM((B2 // 2,), _i32), pltpu.VMEM((B2 // 2,), _i32),
            pltpu.VMEM((B2,), _f32), pltpu.VMEM((B2,), _f32),
            pltpu.SemaphoreType.DMA, pltpu.SemaphoreType.DMA,
            pltpu.SemaphoreType.DMA, pltpu.SemaphoreType.DMA,
            pltpu.SemaphoreType.DMA, pltpu.SemaphoreType.DMA,
        ),
    )(_k2_body)


def kernel(rel_emb_sr, rel_emb_tg, conf_sr, imp_sr, pca_sr,
           conf_tg, imp_tg, pca_tg, head_sr, tail_sr, relation_sr,
           head_tg, tail_tg, relation_tg):
    del rel_emb_sr, rel_emb_tg, relation_sr, relation_tg
    h_sr = head_sr.astype(_i32)
    t_sr = tail_sr.astype(_i32)
    h_tg = head_tg.astype(_i32)
    t_tg = tail_tg.astype(_i32)
    vals_sr, vals_tg, part_sr, part_tg = _k1()(
        conf_sr, imp_sr, pca_sr, h_sr, conf_tg, imp_tg, pca_tg, h_tg)
    adj_sr, adj_tg = _k2()(
        h_sr, t_sr, vals_sr, part_sr, h_tg, t_tg, vals_tg, part_tg)
    return adj_sr, adj_tg


# speedup vs baseline: 1.7420x; 1.7420x over previous
import functools

import jax
import jax.numpy as jnp
from jax import lax
from jax.experimental import pallas as pl
from jax.experimental.pallas import tpu as pltpu
from jax.experimental.pallas import tpu_sc as plsc

N = 50000
E = 1600000
NC = 2
NS = 16
NT = NC * NS
EPT = E // NT
B1 = 2048
NCH1 = 24
B2 = 4096
NCH2 = 12
TAIL = 848
NPAD = 50176
SLICE = NPAD // NS
SV = SLICE // 16
DIAG_T = 25
DIAG_B = N // DIAG_T

_f32 = jnp.float32
_i32 = jnp.int32


@functools.cache
def _mesh():
    return plsc.VectorSubcoreMesh(
        core_axis_name="c", subcore_axis_name="s",
        num_cores=NC, num_subcores=NS)


def _rsqrt_newton(x):
    i = plsc.bitcast(x, _i32)
    y = plsc.bitcast(jnp.int32(0x5F3759DF) - (i >> 1), _f32)
    for _ in range(3):
        y = y * (1.5 - 0.5 * x * y * y)
    return jnp.where(x > 0.0, y, 0.0)


def _issue_first(in_descs, bb):
    for d in in_descs(0, 0, bb):
        d.start()
    for d in in_descs(1, 1, bb):
        d.start()


def _chunk_pipeline(in_descs, out_desc, compute, bb, nchb):
    def issue_in(slot, k, n):
        for d in in_descs(slot, k, n):
            d.start()

    def wait_in(slot, k, n):
        for d in in_descs(slot, k, n):
            d.wait()

    def outer(g, c):
        for b in range(2):
            k = 2 * g + b
            wait_in(b, k, bb)

            @pl.when(k >= 2)
            def _():
                out_desc(b, k - 2, bb).wait()
            compute(b, bb // 16)
            out_desc(b, k, bb).start()

            @pl.when(k < nchb - 2)
            def _():
                issue_in(b, k + 2, bb)

            @pl.when(k == nchb - 2)
            def _():
                issue_in(0, nchb, TAIL)
        return c
    lax.fori_loop(0, nchb // 2, outer, 0)

    wait_in(0, nchb, TAIL)
    out_desc(0, nchb - 2, bb).wait()
    compute(0, TAIL // 16)
    out_desc(0, nchb, TAIL).start()
    out_desc(1, nchb - 1, bb).wait()
    out_desc(0, nchb, TAIL).wait()


def _k1_graph(conf_h, imp_h, pca_h, head_h, vals_h, part_h,
              acc, stage, rs, rowbufs, cbufs, ibufs, pbufs, hbufs, vbufs,
              isems, osems, rsems, wid, sid, cid):
    base = wid * EPT

    def in_descs(slot, k, n):
        off = base + k * B1
        dst = pl.ds(0, n)
        return (
            pltpu.make_async_copy(conf_h.at[pl.ds(off, n)], cbufs[slot].at[dst], isems[slot]),
            pltpu.make_async_copy(imp_h.at[pl.ds(off, n)], ibufs[slot].at[dst], isems[slot]),
            pltpu.make_async_copy(pca_h.at[pl.ds(off, n)], pbufs[slot].at[dst], isems[slot]),
            pltpu.make_async_copy(head_h.at[pl.ds(off, n)], hbufs[slot].at[dst], isems[slot]),
        )

    def out_desc(slot, k, n):
        return pltpu.make_async_copy(
            vbufs[slot].at[pl.ds(0, n)],
            vals_h.at[pl.ds(base + k * B1, n)], osems[slot])

    def compute(slot, nvec):
        cb, ib, pb, hb, vb = (cbufs[slot], ibufs[slot], pbufs[slot],
                              hbufs[slot], vbufs[slot])

        def one(j):
            s = pl.ds(j * 16, 16)
            v = cb[s] * ib[s] * pb[s]
            vb[s] = v
            plsc.addupdate_scatter(acc, [hb[s]], v)

        unroll = 8 if nvec % 8 == 0 else 1

        @functools.partial(plsc.parallel_loop, 0, nvec, unroll=unroll)
        def _(j):
            one(j)

    def edges():
        with jax.named_scope("k1_edges"):
            _chunk_pipeline(in_descs, out_desc, compute, B1, NCH1)

    def stage_fn():
        with jax.named_scope("k1_stage"):
            pltpu.sync_copy(acc, stage.at[pl.ds(sid * NPAD, NPAD)])
            plsc.subcore_barrier()

    def reduce_fn():
        with jax.named_scope("k1_reduce"):
            soff = sid * SLICE

            def zrs_body(i, c):
                for u in range(7):
                    rs[pl.ds((i * 7 + u) * 16, 16)] = jnp.zeros((16,), _f32)
                return c
            lax.fori_loop(0, SV // 7, zrs_body, 0)

            def row_desc(slot, r):
                return pltpu.make_async_copy(
                    stage.at[pl.ds(r * NPAD + soff, SLICE)],
                    rowbufs[slot], rsems[slot])

            row_desc(0, 0).start()
            row_desc(1, 1).start()

            def row_outer(g, c):
                for b in range(2):
                    r = 2 * g + b
                    row_desc(b, r).wait()
                    rb = rowbufs[b]

                    def add_body(i, c2):
                        for u in range(7):
                            t = pl.ds((i * 7 + u) * 16, 16)
                            rs[t] = rs[t] + rb[t]
                        return c2
                    lax.fori_loop(0, SV // 7, add_body, 0)

                    @pl.when(r < NS - 2)
                    def _():
                        row_desc(b, r + 2).start()
                return c
            lax.fori_loop(0, NS // 2, row_outer, 0)
            pltpu.sync_copy(rs, part_h.at[pl.ds(cid * NPAD + soff, SLICE)])
            plsc.subcore_barrier()

    return in_descs, edges, stage_fn, reduce_fn


def _k1_zero_acc(acc):
    with jax.named_scope("k1_zero"):
        def zero_body(i, c):
            for u in range(8):
                acc[pl.ds((i * 8 + u) * 16, 16)] = jnp.zeros((16,), _f32)
            return c
        lax.fori_loop(0, NPAD // 16 // 8, zero_body, 0)


def _k1_body(conf_sr, imp_sr, pca_sr, head_sr,
             conf_tg, imp_tg, pca_tg, head_tg,
             vals_sr, vals_tg, part_sr, part_tg,
             acc, stage, rs, rb0, rb1, cb0, cb1, ib0, ib1, pb0, pb1,
             hb0, hb1, vb0, vb1,
             isem0, isem1, osem0, osem1, rsem0, rsem1):
    sid = lax.axis_index("s")
    cid = lax.axis_index("c")
    wid = sid * NC + cid
    args = (acc, stage, rs, (rb0, rb1), (cb0, cb1), (ib0, ib1), (pb0, pb1),
            (hb0, hb1), (vb0, vb1), (isem0, isem1), (osem0, osem1),
            (rsem0, rsem1), wid, sid, cid)
    in_sr, edges_sr, stage_sr, reduce_sr = _k1_graph(
        conf_sr, imp_sr, pca_sr, head_sr, vals_sr, part_sr, *args)
    in_tg, edges_tg, stage_tg, reduce_tg = _k1_graph(
        conf_tg, imp_tg, pca_tg, head_tg, vals_tg, part_tg, *args)
    _issue_first(in_sr, B1)
    _k1_zero_acc(acc)
    edges_sr()
    stage_sr()
    _issue_first(in_tg, B1)
    _k1_zero_acc(acc)
    reduce_sr()
    edges_tg()
    stage_tg()
    reduce_tg()


@functools.cache
def _k1():
    return functools.partial(
        pl.kernel,
        out_type=(
            jax.ShapeDtypeStruct((E,), _f32),
            jax.ShapeDtypeStruct((E,), _f32),
            jax.ShapeDtypeStruct((NC * NPAD,), _f32),
            jax.ShapeDtypeStruct((NC * NPAD,), _f32),
        ),
        mesh=_mesh(),
        compiler_params=pltpu.CompilerParams(needs_layout_passes=False),
        scratch_types=(
            pltpu.VMEM((NPAD,), _f32),
            pltpu.VMEM_SHARED((NS * NPAD,), _f32),
            pltpu.VMEM((SLICE,), _f32),
            pltpu.VMEM((SLICE,), _f32), pltpu.VMEM((SLICE,), _f32),
            pltpu.VMEM((B1,), _f32), pltpu.VMEM((B1,), _f32),
            pltpu.VMEM((B1,), _f32), pltpu.VMEM((B1,), _f32),
            pltpu.VMEM((B1,), _f32), pltpu.VMEM((B1,), _f32),
            pltpu.VMEM((B1,), _i32), pltpu.VMEM((B1,), _i32),
            pltpu.VMEM((B1,), _f32), pltpu.VMEM((B1,), _f32),
            pltpu.SemaphoreType.DMA, pltpu.SemaphoreType.DMA,
            pltpu.SemaphoreType.DMA, pltpu.SemaphoreType.DMA,
            pltpu.SemaphoreType.DMA, pltpu.SemaphoreType.DMA,
        ),
    )(_k1_body)


def _k2_graph(head_h, tail_h, vals_h, part_h, adj_h,
              d_sh, d_ref, rs, rowbufs, hbufs, tbufs, vbufs, obufs,
              rsems, isems, osems, sid, wid):
    base = wid * EPT

    def in_descs(slot, k, n):
        off = base + k * B2
        dst = pl.ds(0, n)
        return (
            pltpu.make_async_copy(head_h.at[pl.ds(off, n)], hbufs[slot].at[dst], isems[slot]),
            pltpu.make_async_copy(tail_h.at[pl.ds(off, n)], tbufs[slot].at[dst], isems[slot]),
            pltpu.make_async_copy(vals_h.at[pl.ds(off, n)], vbufs[slot].at[dst], isems[slot]),
        )

    _issue_first(in_descs, B2)

    soff = sid * SLICE

    with jax.named_scope("k2_init"):
        def one_body(i, c):
            for u in range(7):
                rs[pl.ds((i * 7 + u) * 16, 16)] = jnp.full((16,), 1.0, _f32)
            return c
        lax.fori_loop(0, SV // 7, one_body, 0)

    def row_desc(slot, r):
        return pltpu.make_async_copy(
            part_h.at[pl.ds(r * NPAD + soff, SLICE)], rowbufs[slot], rsems[slot])

    row_desc(0, 0).start()
    row_desc(1, 1).start()
    with jax.named_scope("k2_reduce"):
        for b in range(NC):
            row_desc(b, b).wait()
            rb = rowbufs[b]

            def add_body(i, c2):
                for u in range(7):
                    t = pl.ds((i * 7 + u) * 16, 16)
                    rs[t] = rs[t] + rb[t]
                return c2
            lax.fori_loop(0, SV // 7, add_body, 0)

    with jax.named_scope("k2_newton"):
        def newton_body(i, c):
            for u in range(4):
                s = pl.ds((i * 4 + u) * 16, 16)
                rowbufs[0][s] = _rsqrt_newton(rs[s])
            return c
        lax.fori_loop(0, SV // 4, newton_body, 0)
        pltpu.sync_copy(rowbufs[0], d_sh.at[pl.ds(soff, SLICE)])
    with jax.named_scope("k2_barrier"):
        plsc.subcore_barrier()

    with jax.named_scope("k2_dcopy"):
        pltpu.sync_copy(d_sh, d_ref)

    def out_desc(slot, k, n):
        return pltpu.make_async_copy(
            obufs[slot].at[pl.ds(0, n)],
            adj_h.at[pl.ds(base + k * B2, n)], osems[slot])

    def compute(slot, nvec):
        hb, tb, vb, ob = hbufs[slot], tbufs[slot], vbufs[slot], obufs[slot]

        def one(j):
            s = pl.ds(j * 16, 16)
            dh = plsc.load_gather(d_ref, [hb[s]])
            dt = plsc.load_gather(d_ref, [tb[s]])
            ob[s] = vb[s] * dh * dt

        unroll = 8 if nvec % 8 == 0 else 1

        @functools.partial(plsc.parallel_loop, 0, nvec, unroll=unroll)
        def _(j):
            one(j)

    with jax.named_scope("k2_gather"):
        _chunk_pipeline(in_descs, out_desc, compute, B2, NCH2)

    with jax.named_scope("k2_diag"):
        @pl.when(wid < DIAG_T)
        def _():
            doff = wid * DIAG_B
            ob = obufs[0]

            def diag_body(i, c):
                for u in range(5):
                    j = i * 5 + u
                    y = d_ref[pl.ds(doff + j * 16, 16)]
                    ob[pl.ds(j * 16, 16)] = y * y
                return c
            lax.fori_loop(0, DIAG_B // 16 // 5, diag_body, 0)
            pltpu.sync_copy(ob.at[pl.ds(0, DIAG_B)],
                            adj_h.at[pl.ds(E + doff, DIAG_B)])


def _k2_body(head_sr, tail_sr, vals_sr, part_sr,
             head_tg, tail_tg, vals_tg, part_tg,
             adj_sr, adj_tg,
             d_sh, d_ref, rs, rb0, rb1, hb0, hb1, tb0, tb1, vb0, vb1,
             ob0, ob1, rsem0, rsem1, isem0, isem1, osem0, osem1):
    sid = lax.axis_index("s")
    wid = sid * NC + lax.axis_index("c")
    args = (d_sh, d_ref, rs, (rb0, rb1), (hb0, hb1), (tb0, tb1), (vb0, vb1),
            (ob0, ob1), (rsem0, rsem1), (isem0, isem1), (osem0, osem1),
            sid, wid)
    _k2_graph(head_sr, tail_sr, vals_sr, part_sr, adj_sr, *args)
    plsc.subcore_barrier()
    _k2_graph(head_tg, tail_tg, vals_tg, part_tg, adj_tg, *args)


@functools.cache
def _k2():
    return functools.partial(
        pl.kernel,
        out_type=(
            jax.ShapeDtypeStruct((E + N,), _f32),
            jax.ShapeDtypeStruct((E + N,), _f32),
        ),
        mesh=_mesh(),
        compiler_params=pltpu.CompilerParams(needs_layout_passes=False),
        scratch_types=(
            pltpu.VMEM_SHARED((NPAD,), _f32),
            pltpu.VMEM((NPAD,), _f32),
            pltpu.VMEM((SLICE,), _f32),
            pltpu.VMEM((SLICE,), _f32), pltpu.VMEM((SLICE,), _f32),
            pltpu.VMEM((B2,), _i32), pltpu.VMEM((B2,), _i32),
            pltpu.VMEM((B2,), _i32), pltpu.VMEM((B2,), _i32),
            pltpu.VMEM((B2,), _f32), pltpu.VMEM((B2,), _f32),
            pltpu.VMEM((B2,), _f32), pltpu.VMEM((B2,), _f32),
            pltpu.SemaphoreType.DMA, pltpu.SemaphoreType.DMA,
            pltpu.SemaphoreType.DMA, pltpu.SemaphoreType.DMA,
            pltpu.SemaphoreType.DMA, pltpu.SemaphoreType.DMA,
        ),
    )(_k2_body)


def kernel(rel_emb_sr, rel_emb_tg, conf_sr, imp_sr, pca_sr,
           conf_tg, imp_tg, pca_tg, head_sr, tail_sr, relation_sr,
           head_tg, tail_tg, relation_tg):
    del rel_emb_sr, rel_emb_tg, relation_sr, relation_tg
    h_sr = head_sr.astype(_i32)
    t_sr = tail_sr.astype(_i32)
    h_tg = head_tg.astype(_i32)
    t_tg = tail_tg.astype(_i32)
    vals_sr, vals_tg, part_sr, part_tg = _k1()(
        conf_sr, imp_sr, pca_sr, h_sr, conf_tg, imp_tg, pca_tg, h_tg)
    adj_sr, adj_tg = _k2()(
        h_sr, t_sr, vals_sr, part_sr, h_tg, t_tg, vals_tg, part_tg)
    return adj_sr, adj_tg
